# baseline (speedup 1.0000x reference)
import jax
import jax.numpy as jnp
from jax.experimental import pallas as pl
from jax.experimental.pallas import tpu as pltpu

H = 56
W = 56
C = 96
N_OP = 4
HW = H * W
PAD = W


def _mixed_op_body(x_ref, wgt_ref, bg_ref, wct_ref, recip_ref,
                   out_ref, g_ref, sw_ref, mw_ref):
    x3 = x_ref[0]
    x2 = x3.reshape(HW, C)

    s1 = jnp.sum(x2.reshape(8, HW // 8, C), axis=0)
    s2 = jnp.sum(s1.reshape(7, W, C), axis=0)
    gm = jnp.sum(s2, axis=0, keepdims=True) * (1.0 / HW)
    grow = jnp.dot(gm.astype(jnp.bfloat16), wgt_ref[...],
                   preferred_element_type=jnp.float32) + bg_ref[...]
    g_ref[0] = grow

    conv2 = jnp.dot(x2.astype(jnp.bfloat16), wct_ref[...],
                    preferred_element_type=jnp.float32)

    ci = jax.lax.broadcasted_iota(jnp.int32, (1, N_OP), 1)
    m1 = jnp.max(grow, axis=1, keepdims=True)
    i1 = jnp.min(jnp.where(grow == m1, ci, N_OP), axis=1, keepdims=True)
    sel1 = ci == i1
    gmsk = jnp.where(sel1, -jnp.inf, grow)
    m2 = jnp.max(gmsk, axis=1, keepdims=True)
    i2 = jnp.min(jnp.where(gmsk == m2, ci, N_OP), axis=1, keepdims=True)
    sel = sel1 | (ci == i2)

    e = jnp.where(sel, jnp.exp(grow - m1), 0.0)
    sm = e / jnp.sum(e, axis=1, keepdims=True)
    w0 = jnp.sum(jnp.where(ci == 0, sm, 0.0), axis=1, keepdims=True)
    w1 = jnp.sum(jnp.where(ci == 1, sm, 0.0), axis=1, keepdims=True)
    w2 = jnp.sum(jnp.where(ci == 2, sm, 0.0), axis=1, keepdims=True)
    w3 = jnp.sum(jnp.where(ci == 3, sm, 0.0), axis=1, keepdims=True)

    out_ref[0] = x3 * w0

    @pl.when(w1[0, 0] > 0.0)
    def _avg():
        zcol = jnp.zeros((H, 1, C), jnp.float32)
        we = jnp.concatenate([zcol, x3[:, : W - 1, :]], axis=1)
        ea = jnp.concatenate([x3[:, 1:, :], zcol], axis=1)
        sw_ref[pl.ds(0, PAD)] = jnp.zeros((PAD, C), jnp.float32)
        sw_ref[pl.ds(PAD, HW)] = (x3 + we + ea).reshape(HW, C)
        sw_ref[pl.ds(PAD + HW, PAD)] = jnp.zeros((PAD, C), jnp.float32)
        shw = sw_ref[pl.ds(0, HW)] + sw_ref[pl.ds(W, HW)] \
            + sw_ref[pl.ds(2 * W, HW)]
        out_ref[0] += ((w1 * recip_ref[...]) * shw).reshape(H, W, C)

    @pl.when(w2[0, 0] > 0.0)
    def _max():
        ninf = jnp.float32(-jnp.inf)
        ncol = jnp.full((H, 1, C), ninf, jnp.float32)
        wem = jnp.concatenate([ncol, x3[:, : W - 1, :]], axis=1)
        eam = jnp.concatenate([x3[:, 1:, :], ncol], axis=1)
        mw_ref[pl.ds(0, PAD)] = jnp.full((PAD, C), ninf, jnp.float32)
        mw_ref[pl.ds(PAD, HW)] = jnp.maximum(
            x3, jnp.maximum(wem, eam)).reshape(HW, C)
        mw_ref[pl.ds(PAD + HW, PAD)] = jnp.full((PAD, C), ninf, jnp.float32)
        mhw = jnp.maximum(mw_ref[pl.ds(0, HW)], jnp.maximum(
            mw_ref[pl.ds(W, HW)], mw_ref[pl.ds(2 * W, HW)]))
        out_ref[0] += (w2 * mhw).reshape(H, W, C)

    @pl.when(w3[0, 0] > 0.0)
    def _conv():
        out_ref[0] += (w3 * conv2).reshape(H, W, C)


@jax.jit
def kernel(x, weights, Wg, bg, Wconv):
    del weights
    B = x.shape[0]
    xt = jnp.transpose(x, (0, 2, 3, 1))

    ih = jax.lax.broadcasted_iota(jnp.int32, (H, W), 0)
    iw = jax.lax.broadcasted_iota(jnp.int32, (H, W), 1)
    ch = jnp.where((ih == 0) | (ih == H - 1), 2.0, 3.0)
    cw = jnp.where((iw == 0) | (iw == W - 1), 2.0, 3.0)
    recip = jnp.broadcast_to((1.0 / (ch * cw)).reshape(HW, 1), (HW, C))

    out_t, g = pl.pallas_call(
        _mixed_op_body,
        grid=(B,),
        in_specs=[
            pl.BlockSpec((1, H, W, C), lambda b: (b, 0, 0, 0)),
            pl.BlockSpec((C, N_OP), lambda b: (0, 0)),
            pl.BlockSpec((1, N_OP), lambda b: (0, 0)),
            pl.BlockSpec((C, C), lambda b: (0, 0)),
            pl.BlockSpec((HW, C), lambda b: (0, 0)),
        ],
        out_specs=[
            pl.BlockSpec((1, H, W, C), lambda b: (b, 0, 0, 0)),
            pl.BlockSpec((1, 1, N_OP), lambda b: (b, 0, 0)),
        ],
        out_shape=[
            jax.ShapeDtypeStruct((B, H, W, C), jnp.float32),
            jax.ShapeDtypeStruct((B, 1, N_OP), jnp.float32),
        ],
        scratch_shapes=[
            pltpu.VMEM((HW + 2 * PAD, C), jnp.float32),
            pltpu.VMEM((HW + 2 * PAD, C), jnp.float32),
        ],
        compiler_params=pltpu.CompilerParams(
            dimension_semantics=("arbitrary",),
        ),
    )(xt, Wg.T.astype(jnp.bfloat16), bg.reshape(1, N_OP),
      Wconv.T.astype(jnp.bfloat16), recip)
    out = jnp.transpose(out_t, (0, 3, 1, 2))
    return (out, g.reshape(B, N_OP))


# speedup vs baseline: 1.0191x; 1.0191x over previous
import jax
import jax.numpy as jnp
from jax.experimental import pallas as pl
from jax.experimental.pallas import tpu as pltpu

H = 56
W = 56
C = 96
N_OP = 4
HW = H * W
PAD = W


def _mixed_op_body(x_ref, wgt_ref, bg_ref, wct_ref, recip_ref,
                   out_ref, g_ref, sw_ref, mw_ref):
    x3 = x_ref[0]
    x2 = x3.reshape(HW, C)

    s1 = jnp.sum(x2.reshape(8, HW // 8, C), axis=0)
    s2 = jnp.sum(s1.reshape(7, W, C), axis=0)
    gm = jnp.sum(s2, axis=0, keepdims=True) * (1.0 / HW)
    grow = jnp.dot(gm.astype(jnp.bfloat16), wgt_ref[...],
                   preferred_element_type=jnp.float32) + bg_ref[...]
    g_ref[0] = grow

    conv2 = jnp.dot(x2.astype(jnp.bfloat16), wct_ref[...],
                    preferred_element_type=jnp.float32)

    ci = jax.lax.broadcasted_iota(jnp.int32, (1, N_OP), 1)
    m1 = jnp.max(grow, axis=1, keepdims=True)
    i1 = jnp.min(jnp.where(grow == m1, ci, N_OP), axis=1, keepdims=True)
    sel1 = ci == i1
    gmsk = jnp.where(sel1, -jnp.inf, grow)
    m2 = jnp.max(gmsk, axis=1, keepdims=True)
    i2 = jnp.min(jnp.where(gmsk == m2, ci, N_OP), axis=1, keepdims=True)
    sel = sel1 | (ci == i2)

    e = jnp.where(sel, jnp.exp(grow - m1), 0.0)
    sm = e / (1.0 + jnp.exp(m2 - m1))
    w0 = jnp.sum(jnp.where(ci == 0, sm, 0.0), axis=1, keepdims=True)
    w1 = jnp.sum(jnp.where(ci == 1, sm, 0.0), axis=1, keepdims=True)
    w2 = jnp.sum(jnp.where(ci == 2, sm, 0.0), axis=1, keepdims=True)
    w3 = jnp.sum(jnp.where(ci == 3, sm, 0.0), axis=1, keepdims=True)

    out_ref[0] = x3 * w0

    @pl.when(w1[0, 0] > 0.0)
    def _avg():
        zcol = jnp.zeros((H, 1, C), jnp.float32)
        we = jnp.concatenate([zcol, x3[:, : W - 1, :]], axis=1)
        ea = jnp.concatenate([x3[:, 1:, :], zcol], axis=1)
        sw_ref[pl.ds(0, PAD)] = jnp.zeros((PAD, C), jnp.float32)
        sw_ref[pl.ds(PAD, HW)] = (x3 + we + ea).reshape(HW, C)
        sw_ref[pl.ds(PAD + HW, PAD)] = jnp.zeros((PAD, C), jnp.float32)
        shw = sw_ref[pl.ds(0, HW)] + sw_ref[pl.ds(W, HW)] \
            + sw_ref[pl.ds(2 * W, HW)]
        out_ref[0] += ((w1 * recip_ref[...]) * shw).reshape(H, W, C)

    @pl.when(w2[0, 0] > 0.0)
    def _max():
        ninf = jnp.float32(-jnp.inf)
        ncol = jnp.full((H, 1, C), ninf, jnp.float32)
        wem = jnp.concatenate([ncol, x3[:, : W - 1, :]], axis=1)
        eam = jnp.concatenate([x3[:, 1:, :], ncol], axis=1)
        mw_ref[pl.ds(0, PAD)] = jnp.full((PAD, C), ninf, jnp.float32)
        mw_ref[pl.ds(PAD, HW)] = jnp.maximum(
            x3, jnp.maximum(wem, eam)).reshape(HW, C)
        mw_ref[pl.ds(PAD + HW, PAD)] = jnp.full((PAD, C), ninf, jnp.float32)
        mhw = jnp.maximum(mw_ref[pl.ds(0, HW)], jnp.maximum(
            mw_ref[pl.ds(W, HW)], mw_ref[pl.ds(2 * W, HW)]))
        out_ref[0] += (w2 * mhw).reshape(H, W, C)

    @pl.when(w3[0, 0] > 0.0)
    def _conv():
        out_ref[0] += (w3 * conv2).reshape(H, W, C)


@jax.jit
def kernel(x, weights, Wg, bg, Wconv):
    del weights
    B = x.shape[0]
    xt = jnp.transpose(x, (0, 2, 3, 1))

    ih = jax.lax.broadcasted_iota(jnp.int32, (H, W), 0)
    iw = jax.lax.broadcasted_iota(jnp.int32, (H, W), 1)
    ch = jnp.where((ih == 0) | (ih == H - 1), 2.0, 3.0)
    cw = jnp.where((iw == 0) | (iw == W - 1), 2.0, 3.0)
    recip = jnp.broadcast_to((1.0 / (ch * cw)).reshape(HW, 1), (HW, C))

    out_t, g = pl.pallas_call(
        _mixed_op_body,
        grid=(B,),
        in_specs=[
            pl.BlockSpec((1, H, W, C), lambda b: (b, 0, 0, 0)),
            pl.BlockSpec((C, N_OP), lambda b: (0, 0)),
            pl.BlockSpec((1, N_OP), lambda b: (0, 0)),
            pl.BlockSpec((C, C), lambda b: (0, 0)),
            pl.BlockSpec((HW, C), lambda b: (0, 0)),
        ],
        out_specs=[
            pl.BlockSpec((1, H, W, C), lambda b: (b, 0, 0, 0)),
            pl.BlockSpec((1, 1, N_OP), lambda b: (b, 0, 0)),
        ],
        out_shape=[
            jax.ShapeDtypeStruct((B, H, W, C), jnp.float32),
            jax.ShapeDtypeStruct((B, 1, N_OP), jnp.float32),
        ],
        scratch_shapes=[
            pltpu.VMEM((HW + 2 * PAD, C), jnp.float32),
            pltpu.VMEM((HW + 2 * PAD, C), jnp.float32),
        ],
        compiler_params=pltpu.CompilerParams(
            dimension_semantics=("arbitrary",),
        ),
    )(xt, Wg.T.astype(jnp.bfloat16), bg.reshape(1, N_OP),
      Wconv.T.astype(jnp.bfloat16), recip)
    out = jnp.transpose(out_t, (0, 3, 1, 2))
    return (out, g.reshape(B, N_OP))


# speedup vs baseline: 1.0195x; 1.0003x over previous
import jax
import jax.numpy as jnp
from jax.experimental import pallas as pl
from jax.experimental.pallas import tpu as pltpu

H = 56
W = 56
C = 96
N_OP = 4
HW = H * W
PAD = W


def _mixed_op_body(x_ref, wgt_ref, bg_ref, wct_ref, recip_ref,
                   out_ref, g_ref, sw_ref, mw_ref):
    x3 = x_ref[0]
    x2 = x3.reshape(HW, C)

    s1 = jnp.sum(x2.reshape(8, HW // 8, C), axis=0)
    s2 = jnp.sum(s1.reshape(7, W, C), axis=0)
    gm = jnp.sum(s2, axis=0, keepdims=True) * (1.0 / HW)
    grow = jnp.dot(gm.astype(jnp.bfloat16), wgt_ref[...],
                   preferred_element_type=jnp.float32) + bg_ref[...]
    g_ref[0] = grow

    conv2 = jnp.dot(x2.astype(jnp.bfloat16), wct_ref[...],
                    preferred_element_type=jnp.float32)

    ci = jax.lax.broadcasted_iota(jnp.int32, (1, N_OP), 1)
    m1 = jnp.max(grow, axis=1, keepdims=True)
    i1 = jnp.min(jnp.where(grow == m1, ci, N_OP), axis=1, keepdims=True)
    sel1 = ci == i1
    gmsk = jnp.where(sel1, -jnp.inf, grow)
    m2 = jnp.max(gmsk, axis=1, keepdims=True)
    i2 = jnp.min(jnp.where(gmsk == m2, ci, N_OP), axis=1, keepdims=True)
    sel = sel1 | (ci == i2)

    e = jnp.where(sel, jnp.exp(grow - m1), 0.0)
    sm = e / (1.0 + jnp.exp(m2 - m1))
    w0 = jnp.sum(jnp.where(ci == 0, sm, 0.0), axis=1, keepdims=True)
    w1 = jnp.sum(jnp.where(ci == 1, sm, 0.0), axis=1, keepdims=True)
    w2 = jnp.sum(jnp.where(ci == 2, sm, 0.0), axis=1, keepdims=True)
    w3 = jnp.sum(jnp.where(ci == 3, sm, 0.0), axis=1, keepdims=True)

    @pl.when(w3[0, 0] > 0.0)
    def _base_with_conv():
        out_ref[0] = x3 * w0 + (w3 * conv2).reshape(H, W, C)

    @pl.when(w3[0, 0] <= 0.0)
    def _base():
        out_ref[0] = x3 * w0

    @pl.when(w1[0, 0] > 0.0)
    def _avg():
        zcol = jnp.zeros((H, 1, C), jnp.float32)
        we = jnp.concatenate([zcol, x3[:, : W - 1, :]], axis=1)
        ea = jnp.concatenate([x3[:, 1:, :], zcol], axis=1)
        sw_ref[pl.ds(0, PAD)] = jnp.zeros((PAD, C), jnp.float32)
        sw_ref[pl.ds(PAD, HW)] = (x3 + we + ea).reshape(HW, C)
        sw_ref[pl.ds(PAD + HW, PAD)] = jnp.zeros((PAD, C), jnp.float32)
        shw = sw_ref[pl.ds(0, HW)] + sw_ref[pl.ds(W, HW)] \
            + sw_ref[pl.ds(2 * W, HW)]
        out_ref[0] += ((w1 * recip_ref[...]) * shw).reshape(H, W, C)

    @pl.when(w2[0, 0] > 0.0)
    def _max():
        ninf = jnp.float32(-jnp.inf)
        ncol = jnp.full((H, 1, C), ninf, jnp.float32)
        wem = jnp.concatenate([ncol, x3[:, : W - 1, :]], axis=1)
        eam = jnp.concatenate([x3[:, 1:, :], ncol], axis=1)
        mw_ref[pl.ds(0, PAD)] = jnp.full((PAD, C), ninf, jnp.float32)
        mw_ref[pl.ds(PAD, HW)] = jnp.maximum(
            x3, jnp.maximum(wem, eam)).reshape(HW, C)
        mw_ref[pl.ds(PAD + HW, PAD)] = jnp.full((PAD, C), ninf, jnp.float32)
        mhw = jnp.maximum(mw_ref[pl.ds(0, HW)], jnp.maximum(
            mw_ref[pl.ds(W, HW)], mw_ref[pl.ds(2 * W, HW)]))
        out_ref[0] += (w2 * mhw).reshape(H, W, C)


@jax.jit
def kernel(x, weights, Wg, bg, Wconv):
    del weights
    B = x.shape[0]
    xt = jnp.transpose(x, (0, 2, 3, 1))

    ih = jax.lax.broadcasted_iota(jnp.int32, (H, W), 0)
    iw = jax.lax.broadcasted_iota(jnp.int32, (H, W), 1)
    ch = jnp.where((ih == 0) | (ih == H - 1), 2.0, 3.0)
    cw = jnp.where((iw == 0) | (iw == W - 1), 2.0, 3.0)
    recip = jnp.broadcast_to((1.0 / (ch * cw)).reshape(HW, 1), (HW, C))

    out_t, g = pl.pallas_call(
        _mixed_op_body,
        grid=(B,),
        in_specs=[
            pl.BlockSpec((1, H, W, C), lambda b: (b, 0, 0, 0)),
            pl.BlockSpec((C, N_OP), lambda b: (0, 0)),
            pl.BlockSpec((1, N_OP), lambda b: (0, 0)),
            pl.BlockSpec((C, C), lambda b: (0, 0)),
            pl.BlockSpec((HW, C), lambda b: (0, 0)),
        ],
        out_specs=[
            pl.BlockSpec((1, H, W, C), lambda b: (b, 0, 0, 0)),
            pl.BlockSpec((1, 1, N_OP), lambda b: (b, 0, 0)),
        ],
        out_shape=[
            jax.ShapeDtypeStruct((B, H, W, C), jnp.float32),
            jax.ShapeDtypeStruct((B, 1, N_OP), jnp.float32),
        ],
        scratch_shapes=[
            pltpu.VMEM((HW + 2 * PAD, C), jnp.float32),
            pltpu.VMEM((HW + 2 * PAD, C), jnp.float32),
        ],
        compiler_params=pltpu.CompilerParams(
            dimension_semantics=("arbitrary",),
        ),
    )(xt, Wg.T.astype(jnp.bfloat16), bg.reshape(1, N_OP),
      Wconv.T.astype(jnp.bfloat16), recip)
    out = jnp.transpose(out_t, (0, 3, 1, 2))
    return (out, g.reshape(B, N_OP))


# speedup vs baseline: 1.0218x; 1.0023x over previous
import jax
import jax.numpy as jnp
from jax.experimental import pallas as pl
from jax.experimental.pallas import tpu as pltpu

H = 56
W = 56
C = 96
N_OP = 4
HW = H * W
PAD = W


def _mixed_op_body(x_ref, wgt_ref, bg_ref, wct_ref, recip_ref,
                   out_ref, g_ref, sw_ref, mw_ref):
    x3 = x_ref[0]
    x2 = x3.reshape(HW, C)

    s1 = jnp.sum(x2.reshape(8, HW // 8, C), axis=0)
    s2 = jnp.sum(s1.reshape(7, W, C), axis=0)
    gm = jnp.sum(s2, axis=0, keepdims=True) * (1.0 / HW)
    grow = jnp.dot(gm.astype(jnp.bfloat16), wgt_ref[...],
                   preferred_element_type=jnp.float32) + bg_ref[...]
    g_ref[0] = grow

    conv2 = jnp.dot(x2.astype(jnp.bfloat16), wct_ref[...],
                    preferred_element_type=jnp.float32)

    ci = jax.lax.broadcasted_iota(jnp.int32, (1, N_OP), 1)
    m1 = jnp.max(grow, axis=1, keepdims=True)
    i1 = jnp.min(jnp.where(grow == m1, ci, N_OP), axis=1, keepdims=True)
    sel1 = ci == i1
    gmsk = jnp.where(sel1, -jnp.inf, grow)
    m2 = jnp.max(gmsk, axis=1, keepdims=True)
    i2 = jnp.min(jnp.where(gmsk == m2, ci, N_OP), axis=1, keepdims=True)
    sel = sel1 | (ci == i2)

    e = jnp.where(sel, jnp.exp(grow - m1), 0.0)
    sm = e / (1.0 + jnp.exp(m2 - m1))
    w0 = jnp.sum(jnp.where(ci == 0, sm, 0.0), axis=1, keepdims=True)
    w1 = jnp.sum(jnp.where(ci == 1, sm, 0.0), axis=1, keepdims=True)
    w2 = jnp.sum(jnp.where(ci == 2, sm, 0.0), axis=1, keepdims=True)
    w3 = jnp.sum(jnp.where(ci == 3, sm, 0.0), axis=1, keepdims=True)

    @pl.when(w3[0, 0] > 0.0)
    def _base_with_conv():
        out_ref[0] = x3 * w0 + (w3 * conv2).reshape(H, W, C)

    @pl.when(w3[0, 0] <= 0.0)
    def _base():
        out_ref[0] = x3 * w0

    @pl.when(w1[0, 0] > 0.0)
    def _avg():
        zcol = jnp.zeros((H, 1, C), jnp.float32)
        we = jnp.concatenate([zcol, x3[:, : W - 1, :]], axis=1)
        ea = jnp.concatenate([x3[:, 1:, :], zcol], axis=1)
        sw_ref[pl.ds(0, PAD)] = jnp.zeros((PAD, C), jnp.float32)
        sw_ref[pl.ds(PAD, HW)] = (x3 + we + ea).reshape(HW, C)
        sw_ref[pl.ds(PAD + HW, PAD)] = jnp.zeros((PAD, C), jnp.float32)
        shw = sw_ref[pl.ds(0, HW)] + sw_ref[pl.ds(W, HW)] \
            + sw_ref[pl.ds(2 * W, HW)]
        out_ref[0] += ((w1 * recip_ref[...]) * shw).reshape(H, W, C)

    @pl.when(w2[0, 0] > 0.0)
    def _max():
        ninf = jnp.float32(-jnp.inf)
        ncol = jnp.full((H, 1, C), ninf, jnp.float32)
        wem = jnp.concatenate([ncol, x3[:, : W - 1, :]], axis=1)
        eam = jnp.concatenate([x3[:, 1:, :], ncol], axis=1)
        mw_ref[pl.ds(0, PAD)] = jnp.full((PAD, C), ninf, jnp.float32)
        mw_ref[pl.ds(PAD, HW)] = jnp.maximum(
            x3, jnp.maximum(wem, eam)).reshape(HW, C)
        mw_ref[pl.ds(PAD + HW, PAD)] = jnp.full((PAD, C), ninf, jnp.float32)
        mhw = jnp.maximum(mw_ref[pl.ds(0, HW)], jnp.maximum(
            mw_ref[pl.ds(W, HW)], mw_ref[pl.ds(2 * W, HW)]))
        out_ref[0] += (w2 * mhw).reshape(H, W, C)


@jax.jit
def kernel(x, weights, Wg, bg, Wconv):
    del weights
    B = x.shape[0]
    xt = jnp.transpose(x, (0, 2, 3, 1))

    ih = jax.lax.broadcasted_iota(jnp.int32, (H, W), 0)
    iw = jax.lax.broadcasted_iota(jnp.int32, (H, W), 1)
    ch = jnp.where((ih == 0) | (ih == H - 1), 2.0, 3.0)
    cw = jnp.where((iw == 0) | (iw == W - 1), 2.0, 3.0)
    recip = jnp.broadcast_to((1.0 / (ch * cw)).reshape(HW, 1), (HW, C))

    out_t, g = pl.pallas_call(
        _mixed_op_body,
        grid=(B,),
        in_specs=[
            pl.BlockSpec((1, H, W, C), lambda b: (b, 0, 0, 0)),
            pl.BlockSpec((C, N_OP), lambda b: (0, 0)),
            pl.BlockSpec((1, N_OP), lambda b: (0, 0)),
            pl.BlockSpec((C, C), lambda b: (0, 0)),
            pl.BlockSpec((HW, C), lambda b: (0, 0)),
        ],
        out_specs=[
            pl.BlockSpec((1, H, W, C), lambda b: (b, 0, 0, 0)),
            pl.BlockSpec((1, 1, N_OP), lambda b: (b, 0, 0)),
        ],
        out_shape=[
            jax.ShapeDtypeStruct((B, H, W, C), jnp.float32),
            jax.ShapeDtypeStruct((B, 1, N_OP), jnp.float32),
        ],
        scratch_shapes=[
            pltpu.VMEM((HW + 2 * PAD, C), jnp.float32),
            pltpu.VMEM((HW + 2 * PAD, C), jnp.float32),
        ],
        compiler_params=pltpu.CompilerParams(
            dimension_semantics=("parallel",),
        ),
    )(xt, Wg.T.astype(jnp.bfloat16), bg.reshape(1, N_OP),
      Wconv.T.astype(jnp.bfloat16), recip)
    out = jnp.transpose(out_t, (0, 3, 1, 2))
    return (out, g.reshape(B, N_OP))
